# baseline (speedup 1.0000x reference)
import functools

import jax
import jax.numpy as jnp
from jax import lax
from jax.experimental import pallas as pl
from jax.experimental.pallas import tpu as pltpu
from jax.experimental.pallas import tpu_sc as plsc

VOCAB = 1000000
DIM = 64
BATCH = 4096
SEQ = 200
NC = 2
NS = 16
NW = NC * NS
CB = BATCH // NW
NCHUNK = SEQ
LOOKAHEAD = 3


def _pidx_vecs(idx_v, k, pidx_v, slot):
    for jb in range(8):
        v = idx_v[k, pl.ds(jb * 16, 16)]
        pidx_v[slot, pl.ds(jb * 16, 16)] = lax.shift_right_logical(v, 1)


def _transpose_chunk(idx_v, k, rows_v, slot, obuf, oslot):
    iota = lax.iota(jnp.int32, 16)
    rowvs = []
    parvs = []
    for jb in range(8):
        rowvs.append(iota + jb * 16)
        v = idx_v[k, pl.ds(jb * 16, 16)]
        parvs.append(lax.shift_left(v & 1, 6))
    rows = rows_v.at[slot]
    out = obuf.at[oslot]

    def body(c, carry):
        for jb in range(8):
            col = parvs[jb] + c
            val = plsc.load_gather(rows, [rowvs[jb], col])
            out[c, pl.ds(jb * 16, 16)] = val
        return carry

    lax.fori_loop(0, DIM, body, 0)


def _emb_body(xt_hbm, table_hbm, out_hbm, idx_v, pidx_v, rows_v, obuf,
              gs0, gs1, gs2, gs3, os0, os1):
    gsems = [gs0, gs1, gs2, gs3]
    osems = [os0, os1]
    wid = lax.axis_index("s") * NC + lax.axis_index("c")
    b0 = wid * CB

    pltpu.sync_copy(xt_hbm.at[pl.ds(0, SEQ), pl.ds(b0, CB)], idx_v)

    def fire_gather(k, slot):
        _pidx_vecs(idx_v, k, pidx_v, slot)
        pltpu.async_copy(
            table_hbm.at[pidx_v.at[slot]], rows_v.at[slot], gsems[slot])

    def wait_gather(k, slot):
        pltpu.make_async_copy(
            table_hbm.at[pidx_v.at[slot]], rows_v.at[slot],
            gsems[slot]).wait()

    def fire_out(k, oslot):
        pltpu.async_copy(
            obuf.at[oslot], out_hbm.at[k, pl.ds(0, DIM), pl.ds(b0, CB)],
            osems[oslot])

    def wait_out(k, oslot):
        pltpu.make_async_copy(
            obuf.at[oslot], out_hbm.at[k, pl.ds(0, DIM), pl.ds(b0, CB)],
            osems[oslot]).wait()

    for u in range(LOOKAHEAD):
        fire_gather(u, u)

    def step(i, carry):
        for u in range(4):
            k = 4 * i + u
            oslot = u & 1
            if u < 2:
                @pl.when(k >= 2)
                def _():
                    wait_out(k - 2, oslot)
            else:
                wait_out(k - 2, oslot)
            wait_gather(k, u)
            _transpose_chunk(idx_v, k, rows_v, u, obuf, oslot)
            fire_out(k, oslot)

            @pl.when(k + LOOKAHEAD < NCHUNK)
            def _():
                fire_gather(k + LOOKAHEAD, (u + LOOKAHEAD) % 4)
        return carry

    lax.fori_loop(0, NCHUNK // 4, step, 0)
    wait_out(NCHUNK - 2, 0)
    wait_out(NCHUNK - 1, 1)


@jax.jit
def _emb(xt, table2):
    mesh = plsc.VectorSubcoreMesh(core_axis_name="c", subcore_axis_name="s")
    kern = functools.partial(
        pl.kernel,
        out_type=jax.ShapeDtypeStruct((SEQ, DIM, BATCH), jnp.float32),
        mesh=mesh,
        scratch_types=[
            pltpu.VMEM((SEQ, CB), jnp.int32),
            pltpu.VMEM((4, CB), jnp.int32),
            pltpu.VMEM((4, CB, 128), jnp.float32),
            pltpu.VMEM((2, DIM, CB), jnp.float32),
            pltpu.SemaphoreType.DMA,
            pltpu.SemaphoreType.DMA,
            pltpu.SemaphoreType.DMA,
            pltpu.SemaphoreType.DMA,
            pltpu.SemaphoreType.DMA,
            pltpu.SemaphoreType.DMA,
        ],
        compiler_params=pltpu.CompilerParams(
            use_tc_tiling_on_sc=True, needs_layout_passes=False),
    )(_emb_body)
    return kern(xt, table2)


def kernel(x, table):
    xt = x.astype(jnp.int32).T
    table2 = table.reshape(VOCAB // 2, 2 * DIM)
    out_t = _emb(xt, table2)
    return out_t.transpose(2, 0, 1)


# speedup vs baseline: 1.0405x; 1.0405x over previous
import functools

import jax
import jax.numpy as jnp
from jax import lax
from jax.experimental import pallas as pl
from jax.experimental.pallas import tpu as pltpu
from jax.experimental.pallas import tpu_sc as plsc

VOCAB = 1000000
DIM = 64
BATCH = 4096
SEQ = 200
NC = 2
NS = 16
NW = NC * NS
CB = BATCH // NW
NCHUNK = SEQ


def _transpose_chunk(rows_v, slot, obuf):
    iota = lax.iota(jnp.int32, 16)
    rowvs = [iota + jb * 16 for jb in range(8)]
    rows = rows_v.at[slot]
    out = obuf.at[slot]

    def body(c, carry):
        col = jnp.full((16,), 0, jnp.int32) + c
        for jb in range(8):
            val = plsc.load_gather(rows, [rowvs[jb], col])
            out[c, pl.ds(jb * 16, 16)] = val
        return carry

    lax.fori_loop(0, DIM, body, 0, unroll=8)


def _emb_body(xt_hbm, table_hbm, out_hbm, idx_v, rows_v, obuf,
              gs0, gs1, os0, os1):
    gsems = [gs0, gs1]
    osems = [os0, os1]
    wid = lax.axis_index("s") * NC + lax.axis_index("c")
    b0 = wid * CB

    pltpu.sync_copy(xt_hbm.at[pl.ds(0, SEQ), pl.ds(b0, CB)], idx_v)

    def fire_gather(k, slot):
        pltpu.async_copy(
            table_hbm.at[idx_v.at[k]], rows_v.at[slot], gsems[slot])

    def wait_gather(k, slot):
        pltpu.make_async_copy(
            table_hbm.at[idx_v.at[k]], rows_v.at[slot], gsems[slot]).wait()

    def fire_out(k, slot):
        pltpu.async_copy(
            obuf.at[slot], out_hbm.at[k, pl.ds(0, DIM), pl.ds(b0, CB)],
            osems[slot])

    def wait_out(k, slot):
        pltpu.make_async_copy(
            obuf.at[slot], out_hbm.at[k, pl.ds(0, DIM), pl.ds(b0, CB)],
            osems[slot]).wait()

    fire_gather(0, 0)
    fire_gather(1, 1)

    def step(i, carry):
        for u in range(2):
            k = 2 * i + u
            wait_gather(k, u)

            @pl.when(k >= 2)
            def _():
                wait_out(k - 2, u)

            _transpose_chunk(rows_v, u, obuf)
            fire_out(k, u)

            @pl.when(k + 2 < NCHUNK)
            def _():
                fire_gather(k + 2, u)
        return carry

    lax.fori_loop(0, NCHUNK // 2, step, 0)
    wait_out(NCHUNK - 2, 0)
    wait_out(NCHUNK - 1, 1)


@jax.jit
def _emb(xt, tablep):
    mesh = plsc.VectorSubcoreMesh(core_axis_name="c", subcore_axis_name="s")
    kern = functools.partial(
        pl.kernel,
        out_type=jax.ShapeDtypeStruct((SEQ, DIM, BATCH), jnp.float32),
        mesh=mesh,
        scratch_types=[
            pltpu.VMEM((SEQ, CB), jnp.int32),
            pltpu.VMEM((2, CB, 128), jnp.float32),
            pltpu.VMEM((2, DIM, CB), jnp.float32),
            pltpu.SemaphoreType.DMA,
            pltpu.SemaphoreType.DMA,
            pltpu.SemaphoreType.DMA,
            pltpu.SemaphoreType.DMA,
        ],
        compiler_params=pltpu.CompilerParams(
            use_tc_tiling_on_sc=True, needs_layout_passes=False),
    )(_emb_body)
    return kern(xt, tablep)


def kernel(x, table):
    xt = x.astype(jnp.int32).T
    tablep = jnp.pad(table, ((0, 0), (0, DIM)))
    out_t = _emb(xt, tablep)
    return out_t.transpose(2, 0, 1)


# speedup vs baseline: 1.2995x; 1.2489x over previous
import functools

import jax
import jax.numpy as jnp
from jax import lax
from jax.experimental import pallas as pl
from jax.experimental.pallas import tpu as pltpu
from jax.experimental.pallas import tpu_sc as plsc

VOCAB = 1000000
DIM = 64
BATCH = 4096
SEQ = 200
NC = 2
NS = 16
NW = NC * NS
CB = BATCH // NW
NCHUNK = SEQ


def _transpose_chunk(rows_v, slot, obuf):
    iota = lax.iota(jnp.int32, 16)
    rowis = [iota + m * 16 for m in range(4)]
    rows = rows_v.at[slot]
    out = obuf.at[slot]

    @plsc.parallel_loop(0, CB, unroll=8)
    def body(j):
        colj = jnp.full((16,), 0, jnp.int32) + j
        vals = [rows[j, pl.ds(m * 16, 16)] for m in range(4)]
        for m in range(4):
            plsc.store_scatter(out, [rowis[m], colj], vals[m])


def _emb_body(xt_hbm, table_hbm, out_hbm, idx_v, rows_v, obuf,
              gs0, gs1, os0, os1):
    gsems = [gs0, gs1]
    osems = [os0, os1]
    wid = lax.axis_index("s") * NC + lax.axis_index("c")
    b0 = wid * CB

    pltpu.sync_copy(xt_hbm.at[pl.ds(0, SEQ), pl.ds(b0, CB)], idx_v)

    def fire_gather(k, slot):
        pltpu.async_copy(
            table_hbm.at[idx_v.at[k]], rows_v.at[slot], gsems[slot])

    def wait_gather(k, slot):
        pltpu.make_async_copy(
            table_hbm.at[idx_v.at[k]], rows_v.at[slot], gsems[slot]).wait()

    def fire_out(k, slot):
        pltpu.async_copy(
            obuf.at[slot], out_hbm.at[k, pl.ds(0, DIM), pl.ds(b0, CB)],
            osems[slot])

    def wait_out(k, slot):
        pltpu.make_async_copy(
            obuf.at[slot], out_hbm.at[k, pl.ds(0, DIM), pl.ds(b0, CB)],
            osems[slot]).wait()

    fire_gather(0, 0)
    fire_gather(1, 1)

    def step(i, carry):
        for u in range(2):
            k = 2 * i + u
            wait_gather(k, u)

            @pl.when(k >= 2)
            def _():
                wait_out(k - 2, u)

            _transpose_chunk(rows_v, u, obuf)
            fire_out(k, u)

            @pl.when(k + 2 < NCHUNK)
            def _():
                fire_gather(k + 2, u)
        return carry

    lax.fori_loop(0, NCHUNK // 2, step, 0)
    wait_out(NCHUNK - 2, 0)
    wait_out(NCHUNK - 1, 1)


@jax.jit
def _emb(xt, tablep):
    mesh = plsc.VectorSubcoreMesh(core_axis_name="c", subcore_axis_name="s")
    kern = functools.partial(
        pl.kernel,
        out_type=jax.ShapeDtypeStruct((SEQ, DIM, BATCH), jnp.float32),
        mesh=mesh,
        scratch_types=[
            pltpu.VMEM((SEQ, CB), jnp.int32),
            pltpu.VMEM((2, CB, 128), jnp.float32),
            pltpu.VMEM((2, DIM, CB), jnp.float32),
            pltpu.SemaphoreType.DMA,
            pltpu.SemaphoreType.DMA,
            pltpu.SemaphoreType.DMA,
            pltpu.SemaphoreType.DMA,
        ],
        compiler_params=pltpu.CompilerParams(
            use_tc_tiling_on_sc=True, needs_layout_passes=False),
    )(_emb_body)
    return kern(xt, tablep)


def kernel(x, table):
    xt = x.astype(jnp.int32).T
    tablep = jnp.pad(table, ((0, 0), (0, DIM)))
    out_t = _emb(xt, tablep)
    return out_t.transpose(2, 0, 1)
